# baseline (speedup 1.0000x reference)
import functools

import jax
import jax.numpy as jnp
from jax import lax
from jax.experimental import pallas as pl
from jax.experimental.pallas import tpu as pltpu
from jax.experimental.pallas import tpu_sc as plsc

N = 10000
E = 160000
T = 12
H = 32

NTILES = 16
CW = 128
GCH = E // CW
ROWS_PT = N // NTILES


def _mm_body(x_ref, w_ref, b_ref, o_ref):
    o_ref[...] = (
        jnp.dot(x_ref[...], w_ref[...], preferred_element_type=jnp.float32)
        + b_ref[...]
    )


def _mm(x, w, b, bm=400):
    m, k = x.shape
    n = w.shape[1]
    assert m % bm == 0, (m, bm)
    return pl.pallas_call(
        _mm_body,
        grid=(m // bm,),
        in_specs=[
            pl.BlockSpec((bm, k), lambda i: (i, 0)),
            pl.BlockSpec((k, n), lambda i: (0, 0)),
            pl.BlockSpec((n,), lambda i: (0,)),
        ],
        out_specs=pl.BlockSpec((bm, n), lambda i: (i, 0)),
        out_shape=jax.ShapeDtypeStruct((m, n), jnp.float32),
    )(x, w, b)


def _gates_ru_body(ct_ref, hcat_ref, w_ref, h_ref, g_ref, u_ref):
    pre = jnp.dot(hcat_ref[...], w_ref[...],
                  preferred_element_type=jnp.float32) + ct_ref[...]
    ru = jax.nn.sigmoid(pre)
    g_ref[...] = ru[:, :32] * h_ref[...]
    u_ref[...] = ru[:, 32:]


def _gates_ru(ct64, hcat, Wh_ru, h, bm=400):
    return pl.pallas_call(
        _gates_ru_body,
        grid=(N // bm,),
        in_specs=[
            pl.BlockSpec((bm, 64), lambda i: (i, 0)),
            pl.BlockSpec((bm, 160), lambda i: (i, 0)),
            pl.BlockSpec((160, 64), lambda i: (0, 0)),
            pl.BlockSpec((bm, 32), lambda i: (i, 0)),
        ],
        out_specs=[pl.BlockSpec((bm, 32), lambda i: (i, 0)),
                   pl.BlockSpec((bm, 32), lambda i: (i, 0))],
        out_shape=[jax.ShapeDtypeStruct((N, 32), jnp.float32),
                   jax.ShapeDtypeStruct((N, 32), jnp.float32)],
    )(ct64, hcat, Wh_ru, h)


def _gates_c_body(ct_ref, gcat_ref, w_ref, u_ref, h_ref, o_ref):
    c = jnp.tanh(jnp.dot(gcat_ref[...], w_ref[...],
                         preferred_element_type=jnp.float32) + ct_ref[...])
    u = u_ref[...]
    o_ref[...] = u * h_ref[...] + (1.0 - u) * c


def _gates_c(ct32, gcat, Wh_c, u, h, bm=400):
    return pl.pallas_call(
        _gates_c_body,
        grid=(N // bm,),
        in_specs=[
            pl.BlockSpec((bm, 32), lambda i: (i, 0)),
            pl.BlockSpec((bm, 160), lambda i: (i, 0)),
            pl.BlockSpec((160, 32), lambda i: (0, 0)),
            pl.BlockSpec((bm, 32), lambda i: (i, 0)),
            pl.BlockSpec((bm, 32), lambda i: (i, 0)),
        ],
        out_specs=pl.BlockSpec((bm, 32), lambda i: (i, 0)),
        out_shape=jax.ShapeDtypeStruct((N, 32), jnp.float32),
    )(ct32, gcat, Wh_c, u, h)


def _decoder_body(h_ref, w1_ref, b1_ref, w2_ref, b2_ref, o_ref):
    z = jax.nn.relu(jnp.dot(h_ref[...], w1_ref[...],
                            preferred_element_type=jnp.float32) + b1_ref[...])
    o_ref[...] = jnp.dot(z, w2_ref[...],
                         preferred_element_type=jnp.float32) + b2_ref[...]


def _decoder(h, W1, b1, W2, b2, bm=400):
    return pl.pallas_call(
        _decoder_body,
        grid=(N // bm,),
        in_specs=[
            pl.BlockSpec((bm, 32), lambda i: (i, 0)),
            pl.BlockSpec((32, 256), lambda i: (0, 0)),
            pl.BlockSpec((256,), lambda i: (0,)),
            pl.BlockSpec((256, 1536), lambda i: (0, 0)),
            pl.BlockSpec((1536,), lambda i: (0,)),
        ],
        out_specs=pl.BlockSpec((bm, 1536), lambda i: (i, 0)),
        out_shape=jax.ShapeDtypeStruct((N, 1536), jnp.float32),
    )(h, W1, b1, W2, b2)


def _make_chain_sc(Tn):
    mesh = plsc.VectorSubcoreMesh(core_axis_name="c", subcore_axis_name="s")

    @functools.partial(
        pl.kernel,
        out_type=[
            jax.ShapeDtypeStruct((Tn, 2, N, H), jnp.float32),
            jax.ShapeDtypeStruct((Tn, 2, N, H), jnp.float32),
        ],
        mesh=mesh,
        compiler_params=pltpu.CompilerParams(use_tc_tiling_on_sc=False),
        scratch_types=[
            pltpu.VMEM((2, 32), jnp.int32),
            pltpu.VMEM((2, CW), jnp.int32),
            pltpu.VMEM((2, CW), jnp.int32),
            pltpu.VMEM((CW,), jnp.float32),
            pltpu.VMEM((CW,), jnp.float32),
            pltpu.VMEM((CW, H), jnp.float32),
            pltpu.VMEM((CW, H), jnp.float32),
            pltpu.VMEM((ROWS_PT, H), jnp.float32),
            pltpu.VMEM_SHARED((N, H), jnp.float32),
            pltpu.VMEM_SHARED((N, H), jnp.float32),
            pltpu.SemaphoreType.DMA,
            pltpu.SemaphoreType.DMA,
            pltpu.SemaphoreType.DMA,
            pltpu.SemaphoreType.DMA,
        ],
    )
    def k(xs_hbm, edata_hbm, wdata_hbm, offs_hbm, out1_hbm, out2_hbm,
          offs_v, ed0, ed1, wd0, wd1, rw0, rw1, acc, bufA, bufB,
          isem0, isem1, gsem0, gsem1):
        cid = lax.axis_index("c")
        sid = lax.axis_index("s")
        rs = sid * ROWS_PT
        basev = jnp.full((16,), 1, jnp.int32) * rs

        pltpu.sync_copy(offs_hbm.at[cid], offs_v)
        lo = offs_v[0, pl.ds(sid, 16)][0]
        hi = offs_v[1, pl.ds(sid, 16)][0]
        n = hi - lo

        ebufs = ((ed0, wd0, isem0, rw0, gsem0), (ed1, wd1, isem1, rw1, gsem1))

        def level(src_buf):
            def zb(i, c):
                acc[i, 0:16] = jnp.zeros((16,), jnp.float32)
                acc[i, 16:32] = jnp.zeros((16,), jnp.float32)
                return c
            lax.fori_loop(0, ROWS_PT, zb, 0)

            @pl.when(n >= 1)
            def _():
                pltpu.async_copy(edata_hbm.at[cid, lo], ed0, isem0)
                pltpu.async_copy(wdata_hbm.at[cid, lo], wd0, isem0)

            @pl.when(n >= 2)
            def _():
                pltpu.async_copy(edata_hbm.at[cid, lo + 1], ed1, isem1)
                pltpu.async_copy(wdata_hbm.at[cid, lo + 1], wd1, isem1)

            @pl.when(n >= 1)
            def _():
                pltpu.make_async_copy(edata_hbm.at[cid, lo], ed0, isem0).wait()
                pltpu.make_async_copy(wdata_hbm.at[cid, lo], wd0, isem0).wait()
                pltpu.async_copy(src_buf.at[ed0.at[0]], rw0, gsem0)

            def body(jj, c):
                for p in (0, 1):
                    ed, wd, isem, rw, gsem = ebufs[p]
                    edo, wdo, isemo, rwo, gsemo = ebufs[1 - p]
                    j = jj * 2 + p

                    @pl.when(j < n)
                    def _():
                        @pl.when(j + 1 < n)
                        def _():
                            pltpu.make_async_copy(
                                edata_hbm.at[cid, lo + j + 1], edo, isemo
                            ).wait()
                            pltpu.make_async_copy(
                                wdata_hbm.at[cid, lo + j + 1], wdo, isemo
                            ).wait()
                            pltpu.async_copy(
                                src_buf.at[edo.at[0]], rwo, gsemo)

                        pltpu.make_async_copy(
                            src_buf.at[ed.at[0]], rw, gsem).wait()

                        for g in range(8):
                            sl = pl.ds(g * 16, 16)
                            dvec = ed[1, sl] - basev
                            inr = jnp.logical_and(dvec >= 0, dvec < ROWS_PT)
                            wvec = wd[sl]
                            wm = jnp.where(inr, wvec, 0.0)
                            dcl = jnp.clip(dvec, 0, ROWS_PT - 1)
                            for e in range(16):
                                b = wm.at[jnp.full((16,), e, jnp.int32)].get(
                                    mode="promise_in_bounds")
                                dloc = dcl[e]
                                r = g * 16 + e
                                plsc.addupdate(acc.at[dloc, 0:16],
                                               rw[r, 0:16] * b)
                                plsc.addupdate(acc.at[dloc, 16:32],
                                               rw[r, 16:32] * b)

                        @pl.when(j + 2 < n)
                        def _():
                            pltpu.async_copy(
                                edata_hbm.at[cid, lo + j + 2], ed, isem)
                            pltpu.async_copy(
                                wdata_hbm.at[cid, lo + j + 2], wd, isem)
                return c
            lax.fori_loop(0, (n + 1) // 2, body, 0)

        def per_t(t, c):
            pltpu.sync_copy(xs_hbm.at[t, pl.ds(rs, ROWS_PT)],
                            bufA.at[pl.ds(rs, ROWS_PT)])
            plsc.subcore_barrier()
            level(bufA)
            pltpu.sync_copy(acc, bufB.at[pl.ds(rs, ROWS_PT)])
            pltpu.sync_copy(acc, out1_hbm.at[t, cid, pl.ds(rs, ROWS_PT)])
            plsc.subcore_barrier()
            level(bufB)
            pltpu.sync_copy(acc, out2_hbm.at[t, cid, pl.ds(rs, ROWS_PT)])
            return c
        lax.fori_loop(0, Tn, per_t, 0)

    return k


_chain_sc_T = _make_chain_sc(T)
_chain_sc_1 = _make_chain_sc(1)


def _chain_cat(xs, edata, wdata, offs, kfun):
    o1, o2 = kfun(xs, edata, wdata, offs)
    return jnp.concatenate(
        [xs, o1[:, 0], o2[:, 0], o1[:, 1], o2[:, 1]], axis=-1)


def _split_w(W):
    Wr = W.reshape(5, 64, H)
    return Wr[:, :32].reshape(160, H), Wr[:, 32:].reshape(160, H)


def _edge_layout(gidx, sidx, w):
    order = jnp.argsort(sidx)
    gs = gidx[order]
    ss = sidx[order]
    ws = w[order]
    edata = jnp.stack([gs.reshape(GCH, CW), ss.reshape(GCH, CW)], axis=1)
    wdata = ws.reshape(GCH, CW)
    bounds = jnp.arange(NTILES + 1, dtype=jnp.int32) * ROWS_PT
    pos = jnp.searchsorted(ss, bounds, side="left").astype(jnp.int32)
    lo_e, hi_e = pos[:-1], pos[1:]
    ch_lo = lo_e // CW
    ch_hi = jnp.where(hi_e > lo_e, -(-hi_e // CW), ch_lo)
    pad = jnp.zeros((NTILES,), jnp.int32)
    offs = jnp.stack([jnp.concatenate([ch_lo, pad]),
                      jnp.concatenate([ch_hi, pad])])
    return edata, wdata, offs


def kernel(x, edge_index, edge_weight, W_enc, b_enc, Wr, br, Wu, bu, Wc, bc, W1, b1, W2, b2):
    src = edge_index[0]
    dst = edge_index[1]
    deg_f = jnp.zeros((N,), jnp.float32).at[dst].add(edge_weight)
    deg_b = jnp.zeros((N,), jnp.float32).at[src].add(edge_weight)
    wf = edge_weight / jnp.where(deg_f == 0, 1.0, deg_f)[dst]
    wb = edge_weight / jnp.where(deg_b == 0, 1.0, deg_b)[src]

    ed_f, wd_f, offs_f = _edge_layout(src, dst, wf)
    ed_b, wd_b, offs_b = _edge_layout(dst, src, wb)
    edata = jnp.stack([ed_f, ed_b])
    wdata = jnp.stack([wd_f, wd_b])
    offs = jnp.stack([offs_f, offs_b])

    Wx_r, Wh_r = _split_w(Wr)
    Wx_u, Wh_u = _split_w(Wu)
    Wx_c, Wh_c = _split_w(Wc)
    Wx_all = jnp.concatenate([Wx_r, Wx_u, Wx_c], axis=1)
    b_all = jnp.concatenate([br, bu, bc])
    Wh_ru = jnp.concatenate([Wh_r, Wh_u], axis=1)

    xe = _mm(x[0].reshape(T * N, -1), W_enc, b_enc).reshape(T, N, H)

    xcat = _chain_cat(xe, edata, wdata, offs, _chain_sc_T)
    ct = _mm(xcat.reshape(T * N, 160), Wx_all, b_all).reshape(T, N, 96)

    h = jnp.zeros((N, H), jnp.float32)
    for t in range(T):
        hcat = _chain_cat(h[None], edata, wdata, offs, _chain_sc_1)[0]
        g, u = _gates_ru(ct[t, :, :64], hcat, Wh_ru, h)
        gcat = _chain_cat(g[None], edata, wdata, offs, _chain_sc_1)[0]
        h = _gates_c(ct[t, :, 64:], gcat, Wh_c, u, h)

    z = _decoder(h, W1, b1, W2, b2)
    out = z.reshape(1, N, 12, 128).transpose(0, 2, 1, 3)
    return out


# speedup vs baseline: 2.2236x; 2.2236x over previous
import functools

import jax
import jax.numpy as jnp
from jax import lax
from jax.experimental import pallas as pl
from jax.experimental.pallas import tpu as pltpu
from jax.experimental.pallas import tpu_sc as plsc

N = 10000
E = 160000
T = 12
H = 32

NTILES = 16
CW = 128
CHUNKS = 80
NPAIRS = CHUNKS // 2
EPT = CHUNKS * CW
EP = EPT * NTILES
ROWS_PT = N // NTILES


def _mm_body(x_ref, w_ref, b_ref, o_ref):
    o_ref[...] = (
        jnp.dot(x_ref[...], w_ref[...], preferred_element_type=jnp.float32)
        + b_ref[...]
    )


def _mm(x, w, b, bm=400):
    m, k = x.shape
    n = w.shape[1]
    assert m % bm == 0, (m, bm)
    return pl.pallas_call(
        _mm_body,
        grid=(m // bm,),
        in_specs=[
            pl.BlockSpec((bm, k), lambda i: (i, 0)),
            pl.BlockSpec((k, n), lambda i: (0, 0)),
            pl.BlockSpec((n,), lambda i: (0,)),
        ],
        out_specs=pl.BlockSpec((bm, n), lambda i: (i, 0)),
        out_shape=jax.ShapeDtypeStruct((m, n), jnp.float32),
    )(x, w, b)


def _gates_ru_body(ct_ref, hcat_ref, w_ref, h_ref, g_ref, u_ref):
    pre = jnp.dot(hcat_ref[...], w_ref[...],
                  preferred_element_type=jnp.float32) + ct_ref[...]
    ru = jax.nn.sigmoid(pre)
    g_ref[...] = ru[:, :32] * h_ref[...]
    u_ref[...] = ru[:, 32:]


def _gates_ru(ct64, hcat, Wh_ru, h, bm=400):
    return pl.pallas_call(
        _gates_ru_body,
        grid=(N // bm,),
        in_specs=[
            pl.BlockSpec((bm, 64), lambda i: (i, 0)),
            pl.BlockSpec((bm, 160), lambda i: (i, 0)),
            pl.BlockSpec((160, 64), lambda i: (0, 0)),
            pl.BlockSpec((bm, 32), lambda i: (i, 0)),
        ],
        out_specs=[pl.BlockSpec((bm, 32), lambda i: (i, 0)),
                   pl.BlockSpec((bm, 32), lambda i: (i, 0))],
        out_shape=[jax.ShapeDtypeStruct((N, 32), jnp.float32),
                   jax.ShapeDtypeStruct((N, 32), jnp.float32)],
    )(ct64, hcat, Wh_ru, h)


def _gates_c_body(ct_ref, gcat_ref, w_ref, u_ref, h_ref, o_ref):
    c = jnp.tanh(jnp.dot(gcat_ref[...], w_ref[...],
                         preferred_element_type=jnp.float32) + ct_ref[...])
    u = u_ref[...]
    o_ref[...] = u * h_ref[...] + (1.0 - u) * c


def _gates_c(ct32, gcat, Wh_c, u, h, bm=400):
    return pl.pallas_call(
        _gates_c_body,
        grid=(N // bm,),
        in_specs=[
            pl.BlockSpec((bm, 32), lambda i: (i, 0)),
            pl.BlockSpec((bm, 160), lambda i: (i, 0)),
            pl.BlockSpec((160, 32), lambda i: (0, 0)),
            pl.BlockSpec((bm, 32), lambda i: (i, 0)),
            pl.BlockSpec((bm, 32), lambda i: (i, 0)),
        ],
        out_specs=pl.BlockSpec((bm, 32), lambda i: (i, 0)),
        out_shape=jax.ShapeDtypeStruct((N, 32), jnp.float32),
    )(ct32, gcat, Wh_c, u, h)


def _decoder_body(h_ref, w1_ref, b1_ref, w2_ref, b2_ref, o_ref):
    z = jax.nn.relu(jnp.dot(h_ref[...], w1_ref[...],
                            preferred_element_type=jnp.float32) + b1_ref[...])
    o_ref[...] = jnp.dot(z, w2_ref[...],
                         preferred_element_type=jnp.float32) + b2_ref[...]


def _decoder(h, W1, b1, W2, b2, bm=400):
    return pl.pallas_call(
        _decoder_body,
        grid=(N // bm,),
        in_specs=[
            pl.BlockSpec((bm, 32), lambda i: (i, 0)),
            pl.BlockSpec((32, 256), lambda i: (0, 0)),
            pl.BlockSpec((256,), lambda i: (0,)),
            pl.BlockSpec((256, 1536), lambda i: (0, 0)),
            pl.BlockSpec((1536,), lambda i: (0,)),
        ],
        out_specs=pl.BlockSpec((bm, 1536), lambda i: (i, 0)),
        out_shape=jax.ShapeDtypeStruct((N, 1536), jnp.float32),
    )(h, W1, b1, W2, b2)


def _make_chain_sc(Tn):
    mesh = plsc.VectorSubcoreMesh(core_axis_name="c", subcore_axis_name="s")

    @functools.partial(
        pl.kernel,
        out_type=[
            jax.ShapeDtypeStruct((Tn, 2, N, H), jnp.float32),
            jax.ShapeDtypeStruct((Tn, 2, N, H), jnp.float32),
        ],
        mesh=mesh,
        compiler_params=pltpu.CompilerParams(use_tc_tiling_on_sc=False),
        scratch_types=[
            pltpu.VMEM((CHUNKS, CW), jnp.int32),
            pltpu.VMEM((CHUNKS, CW), jnp.int32),
            pltpu.VMEM((CHUNKS, CW), jnp.float32),
            pltpu.VMEM((CW, H), jnp.float32),
            pltpu.VMEM((CW, H), jnp.float32),
            pltpu.VMEM((CW, H), jnp.float32),
            pltpu.VMEM((CW, H), jnp.float32),
            pltpu.VMEM((ROWS_PT, H), jnp.float32),
            pltpu.VMEM_SHARED((N, H), jnp.float32),
            pltpu.VMEM_SHARED((N, H), jnp.float32),
            pltpu.SemaphoreType.DMA,
            pltpu.SemaphoreType.DMA,
            pltpu.SemaphoreType.DMA,
            pltpu.SemaphoreType.DMA,
        ],
    )
    def k(xs_hbm, gi_hbm, si_hbm, we_hbm, out1_hbm, out2_hbm,
          gi_v, si_v, w_v, mg0, mg1, ms0, ms1, zblk, bufA, bufB,
          gsem0, gsem1, ssem0, ssem1):
        cid = lax.axis_index("c")
        sid = lax.axis_index("s")
        rs = sid * ROWS_PT

        pltpu.sync_copy(gi_hbm.at[cid, sid], gi_v)
        pltpu.sync_copy(si_hbm.at[cid, sid], si_v)
        pltpu.sync_copy(we_hbm.at[cid, sid], w_v)

        def zb(i, c):
            zblk[i, 0:16] = jnp.zeros((16,), jnp.float32)
            zblk[i, 16:32] = jnp.zeros((16,), jnp.float32)
            return c
        lax.fori_loop(0, ROWS_PT, zb, 0)

        bufs = ((mg0, ms0, gsem0, ssem0), (mg1, ms1, gsem1, ssem1))

        def level(src_buf, acc_buf):
            pltpu.async_copy(src_buf.at[gi_v.at[0]], mg0, gsem0)
            pltpu.async_copy(src_buf.at[gi_v.at[1]], mg1, gsem1)

            def pair(pr, c):
                j0 = pr * 2
                for p in (0, 1):
                    j = j0 + p
                    mg, ms, gs, ss = bufs[p]
                    pltpu.make_async_copy(src_buf.at[gi_v.at[j]], mg, gs).wait()

                    @pl.when(pr >= 1)
                    def _():
                        pltpu.make_async_copy(
                            ms, acc_buf.at[si_v.at[j - 2]], ss).wait()

                    for g in range(8):
                        wv = w_v[j, g * 16:(g + 1) * 16]
                        for e in range(16):
                            b = wv.at[jnp.full((16,), e, jnp.int32)].get(
                                mode="promise_in_bounds")
                            r = g * 16 + e
                            ms[r, 0:16] = mg[r, 0:16] * b
                            ms[r, 16:32] = mg[r, 16:32] * b

                    pltpu.async_copy(ms, acc_buf.at[si_v.at[j]], ss, add=True)

                    @pl.when(pr < NPAIRS - 1)
                    def _():
                        pltpu.async_copy(src_buf.at[gi_v.at[j + 2]], mg, gs)
                return c
            lax.fori_loop(0, NPAIRS, pair, 0)
            pltpu.make_async_copy(
                ms0, acc_buf.at[si_v.at[CHUNKS - 2]], ssem0).wait()
            pltpu.make_async_copy(
                ms1, acc_buf.at[si_v.at[CHUNKS - 1]], ssem1).wait()

        def per_t(t, c):
            pltpu.sync_copy(xs_hbm.at[t, pl.ds(rs, ROWS_PT)],
                            bufA.at[pl.ds(rs, ROWS_PT)])
            pltpu.sync_copy(zblk, bufB.at[pl.ds(rs, ROWS_PT)])
            plsc.subcore_barrier()
            level(bufA, bufB)
            plsc.subcore_barrier()
            pltpu.sync_copy(bufB.at[pl.ds(rs, ROWS_PT)],
                            out1_hbm.at[t, cid, pl.ds(rs, ROWS_PT)])
            pltpu.sync_copy(zblk, bufA.at[pl.ds(rs, ROWS_PT)])
            plsc.subcore_barrier()
            level(bufB, bufA)
            plsc.subcore_barrier()
            pltpu.sync_copy(bufA.at[pl.ds(rs, ROWS_PT)],
                            out2_hbm.at[t, cid, pl.ds(rs, ROWS_PT)])
            return c
        lax.fori_loop(0, Tn, per_t, 0)

    return k


_chain_sc_T = _make_chain_sc(T)
_chain_sc_1 = _make_chain_sc(1)


def _chain_cat(xs, gi, si, we, kfun):
    o1, o2 = kfun(xs, gi, si, we)
    return jnp.concatenate(
        [xs, o1[:, 0], o2[:, 0], o1[:, 1], o2[:, 1]], axis=-1)


def _split_w(W):
    Wr = W.reshape(5, 64, H)
    return Wr[:, :32].reshape(160, H), Wr[:, 32:].reshape(160, H)


def kernel(x, edge_index, edge_weight, W_enc, b_enc, Wr, br, Wu, bu, Wc, bc, W1, b1, W2, b2):
    src = edge_index[0]
    dst = edge_index[1]
    deg_f = jnp.zeros((N,), jnp.float32).at[dst].add(edge_weight)
    deg_b = jnp.zeros((N,), jnp.float32).at[src].add(edge_weight)
    wf = edge_weight / jnp.where(deg_f == 0, 1.0, deg_f)[dst]
    wb = edge_weight / jnp.where(deg_b == 0, 1.0, deg_b)[src]

    pad = EP - E
    zi = jnp.zeros((pad,), jnp.int32)
    zf = jnp.zeros((pad,), jnp.float32)
    of = jnp.argsort(dst)
    ob = jnp.argsort(src)
    src_f = jnp.concatenate([src[of], zi])
    dst_f = jnp.concatenate([dst[of], zi])
    wf_s = jnp.concatenate([wf[of], zf])
    src_b = jnp.concatenate([src[ob], zi])
    dst_b = jnp.concatenate([dst[ob], zi])
    wb_s = jnp.concatenate([wb[ob], zf])
    gi = jnp.stack([src_f, dst_b]).reshape(2, NTILES, CHUNKS, CW)
    si = jnp.stack([dst_f, src_b]).reshape(2, NTILES, CHUNKS, CW)
    we = jnp.stack([wf_s, wb_s]).reshape(2, NTILES, CHUNKS, CW)

    Wx_r, Wh_r = _split_w(Wr)
    Wx_u, Wh_u = _split_w(Wu)
    Wx_c, Wh_c = _split_w(Wc)
    Wx_all = jnp.concatenate([Wx_r, Wx_u, Wx_c], axis=1)
    b_all = jnp.concatenate([br, bu, bc])
    Wh_ru = jnp.concatenate([Wh_r, Wh_u], axis=1)

    xe = _mm(x[0].reshape(T * N, -1), W_enc, b_enc).reshape(T, N, H)

    xcat = _chain_cat(xe, gi, si, we, _chain_sc_T)
    ct = _mm(xcat.reshape(T * N, 160), Wx_all, b_all).reshape(T, N, 96)

    h = jnp.zeros((N, H), jnp.float32)
    for t in range(T):
        hcat = _chain_cat(h[None], gi, si, we, _chain_sc_1)[0]
        g, u = _gates_ru(ct[t, :, :64], hcat, Wh_ru, h)
        gcat = _chain_cat(g[None], gi, si, we, _chain_sc_1)[0]
        h = _gates_c(ct[t, :, 64:], gcat, Wh_c, u, h)

    z = _decoder(h, W1, b1, W2, b2)
    out = z.reshape(1, N, 12, 128).transpose(0, 2, 1, 3)
    return out


# speedup vs baseline: 2.4474x; 1.1007x over previous
import functools

import jax
import jax.numpy as jnp
from jax import lax
from jax.experimental import pallas as pl
from jax.experimental.pallas import tpu as pltpu
from jax.experimental.pallas import tpu_sc as plsc

N = 10000
E = 160000
T = 12
H = 32

NTILES = 16
CW = 128
CHUNKS = 80
NPAIRS = CHUNKS // 2
EPT = CHUNKS * CW
EP = EPT * NTILES
ROWS_PT = N // NTILES


def _mm_body(x_ref, w_ref, b_ref, o_ref):
    o_ref[...] = (
        jnp.dot(x_ref[...], w_ref[...], preferred_element_type=jnp.float32)
        + b_ref[...]
    )


def _mm(x, w, b, bm=400):
    m, k = x.shape
    n = w.shape[1]
    assert m % bm == 0, (m, bm)
    return pl.pallas_call(
        _mm_body,
        grid=(m // bm,),
        in_specs=[
            pl.BlockSpec((bm, k), lambda i: (i, 0)),
            pl.BlockSpec((k, n), lambda i: (0, 0)),
            pl.BlockSpec((n,), lambda i: (0,)),
        ],
        out_specs=pl.BlockSpec((bm, n), lambda i: (i, 0)),
        out_shape=jax.ShapeDtypeStruct((m, n), jnp.float32),
    )(x, w, b)


def _gates_ru_body(ct_ref, hcat_ref, w_ref, h_ref, g_ref, u_ref):
    pre = jnp.dot(hcat_ref[...], w_ref[...],
                  preferred_element_type=jnp.float32) + ct_ref[...]
    ru = jax.nn.sigmoid(pre)
    g_ref[...] = ru[:, :32] * h_ref[...]
    u_ref[...] = ru[:, 32:]


def _gates_ru(ct64, hcat, Wh_ru, h, bm=400):
    return pl.pallas_call(
        _gates_ru_body,
        grid=(N // bm,),
        in_specs=[
            pl.BlockSpec((bm, 64), lambda i: (i, 0)),
            pl.BlockSpec((bm, 160), lambda i: (i, 0)),
            pl.BlockSpec((160, 64), lambda i: (0, 0)),
            pl.BlockSpec((bm, 32), lambda i: (i, 0)),
        ],
        out_specs=[pl.BlockSpec((bm, 32), lambda i: (i, 0)),
                   pl.BlockSpec((bm, 32), lambda i: (i, 0))],
        out_shape=[jax.ShapeDtypeStruct((N, 32), jnp.float32),
                   jax.ShapeDtypeStruct((N, 32), jnp.float32)],
    )(ct64, hcat, Wh_ru, h)


def _gates_c_body(ct_ref, gcat_ref, w_ref, u_ref, h_ref, o_ref):
    c = jnp.tanh(jnp.dot(gcat_ref[...], w_ref[...],
                         preferred_element_type=jnp.float32) + ct_ref[...])
    u = u_ref[...]
    o_ref[...] = u * h_ref[...] + (1.0 - u) * c


def _gates_c(ct32, gcat, Wh_c, u, h, bm=400):
    return pl.pallas_call(
        _gates_c_body,
        grid=(N // bm,),
        in_specs=[
            pl.BlockSpec((bm, 32), lambda i: (i, 0)),
            pl.BlockSpec((bm, 160), lambda i: (i, 0)),
            pl.BlockSpec((160, 32), lambda i: (0, 0)),
            pl.BlockSpec((bm, 32), lambda i: (i, 0)),
            pl.BlockSpec((bm, 32), lambda i: (i, 0)),
        ],
        out_specs=pl.BlockSpec((bm, 32), lambda i: (i, 0)),
        out_shape=jax.ShapeDtypeStruct((N, 32), jnp.float32),
    )(ct32, gcat, Wh_c, u, h)


def _decoder_body(h_ref, w1_ref, b1_ref, w2_ref, b2_ref, o_ref):
    z = jax.nn.relu(jnp.dot(h_ref[...], w1_ref[...],
                            preferred_element_type=jnp.float32) + b1_ref[...])
    o_ref[...] = jnp.dot(z, w2_ref[...],
                         preferred_element_type=jnp.float32) + b2_ref[...]


def _decoder(h, W1, b1, W2, b2, bm=400):
    return pl.pallas_call(
        _decoder_body,
        grid=(N // bm,),
        in_specs=[
            pl.BlockSpec((bm, 32), lambda i: (i, 0)),
            pl.BlockSpec((32, 256), lambda i: (0, 0)),
            pl.BlockSpec((256,), lambda i: (0,)),
            pl.BlockSpec((256, 1536), lambda i: (0, 0)),
            pl.BlockSpec((1536,), lambda i: (0,)),
        ],
        out_specs=pl.BlockSpec((bm, 1536), lambda i: (i, 0)),
        out_shape=jax.ShapeDtypeStruct((N, 1536), jnp.float32),
    )(h, W1, b1, W2, b2)


def _make_chain_sc(Tn):
    mesh = plsc.VectorSubcoreMesh(core_axis_name="c", subcore_axis_name="s")

    @functools.partial(
        pl.kernel,
        out_type=[
            jax.ShapeDtypeStruct((Tn, 2, N, H), jnp.float32),
            jax.ShapeDtypeStruct((Tn, 2, N, H), jnp.float32),
        ],
        mesh=mesh,
        compiler_params=pltpu.CompilerParams(use_tc_tiling_on_sc=False),
        scratch_types=[
            pltpu.VMEM((CHUNKS, CW), jnp.int32),
            pltpu.VMEM((CHUNKS, CW), jnp.int32),
            pltpu.VMEM((CHUNKS, CW), jnp.float32),
            pltpu.VMEM((CW, H), jnp.float32),
            pltpu.VMEM((CW, H), jnp.float32),
            pltpu.VMEM((CW, H), jnp.float32),
            pltpu.VMEM((CW, H), jnp.float32),
            pltpu.VMEM((ROWS_PT, H), jnp.float32),
            pltpu.VMEM_SHARED((N, H), jnp.float32),
            pltpu.VMEM_SHARED((N, H), jnp.float32),
            pltpu.SemaphoreType.DMA,
            pltpu.SemaphoreType.DMA,
            pltpu.SemaphoreType.DMA,
            pltpu.SemaphoreType.DMA,
        ],
    )
    def k(xs_hbm, gi_hbm, si_hbm, we_hbm, out1_hbm, out2_hbm,
          gi_v, si_v, w_v, mg0, mg1, ms0, ms1, zblk, bufA, bufB,
          gsem0, gsem1, ssem0, ssem1):
        cid = lax.axis_index("c")
        sid = lax.axis_index("s")
        rs = sid * ROWS_PT

        pltpu.sync_copy(gi_hbm.at[cid, sid], gi_v)
        pltpu.sync_copy(si_hbm.at[cid, sid], si_v)
        pltpu.sync_copy(we_hbm.at[cid, sid], w_v)

        def zb(i, c):
            zblk[i, 0:16] = jnp.zeros((16,), jnp.float32)
            zblk[i, 16:32] = jnp.zeros((16,), jnp.float32)
            return c
        lax.fori_loop(0, ROWS_PT, zb, 0)

        bufs = ((mg0, ms0, gsem0, ssem0), (mg1, ms1, gsem1, ssem1))

        def level(src_buf, acc_buf):
            pltpu.async_copy(src_buf.at[gi_v.at[0]], mg0, gsem0)
            pltpu.async_copy(src_buf.at[gi_v.at[1]], mg1, gsem1)

            def pair(pr, c):
                j0 = pr * 2
                for p in (0, 1):
                    j = j0 + p
                    mg, ms, gs, ss = bufs[p]
                    pltpu.make_async_copy(src_buf.at[gi_v.at[j]], mg, gs).wait()

                    @pl.when(pr >= 1)
                    def _():
                        pltpu.make_async_copy(
                            ms, acc_buf.at[si_v.at[j - 2]], ss).wait()

                    for g in range(8):
                        wv = w_v[j, g * 16:(g + 1) * 16]
                        for e in range(16):
                            b = wv.at[jnp.full((16,), e, jnp.int32)].get(
                                mode="promise_in_bounds")
                            r = g * 16 + e
                            ms[r, 0:16] = mg[r, 0:16] * b
                            ms[r, 16:32] = mg[r, 16:32] * b

                    pltpu.async_copy(ms, acc_buf.at[si_v.at[j]], ss, add=True)

                    @pl.when(pr < NPAIRS - 1)
                    def _():
                        pltpu.async_copy(src_buf.at[gi_v.at[j + 2]], mg, gs)
                return c
            lax.fori_loop(0, NPAIRS, pair, 0)
            pltpu.make_async_copy(
                ms0, acc_buf.at[si_v.at[CHUNKS - 2]], ssem0).wait()
            pltpu.make_async_copy(
                ms1, acc_buf.at[si_v.at[CHUNKS - 1]], ssem1).wait()

        def per_t(t, c):
            pltpu.sync_copy(xs_hbm.at[t, pl.ds(rs, ROWS_PT)],
                            bufA.at[pl.ds(rs, ROWS_PT)])
            pltpu.sync_copy(zblk, bufB.at[pl.ds(rs, ROWS_PT)])
            plsc.subcore_barrier()
            level(bufA, bufB)
            plsc.subcore_barrier()
            pltpu.sync_copy(bufB.at[pl.ds(rs, ROWS_PT)],
                            out1_hbm.at[t, cid, pl.ds(rs, ROWS_PT)])
            pltpu.sync_copy(zblk, bufA.at[pl.ds(rs, ROWS_PT)])
            plsc.subcore_barrier()
            level(bufB, bufA)
            plsc.subcore_barrier()
            pltpu.sync_copy(bufA.at[pl.ds(rs, ROWS_PT)],
                            out2_hbm.at[t, cid, pl.ds(rs, ROWS_PT)])
            return c
        lax.fori_loop(0, Tn, per_t, 0)

    return k


_chain_sc_T = _make_chain_sc(T)
_chain_sc_1 = _make_chain_sc(1)


def _chain_cat(xs, gi, si, we, kfun):
    o1, o2 = kfun(xs, gi, si, we)
    return jnp.concatenate(
        [xs, o1[:, 0], o2[:, 0], o1[:, 1], o2[:, 1]], axis=-1)


def _split_w(W):
    Wr = W.reshape(5, 64, H)
    return Wr[:, :32].reshape(160, H), Wr[:, 32:].reshape(160, H)


def kernel(x, edge_index, edge_weight, W_enc, b_enc, Wr, br, Wu, bu, Wc, bc, W1, b1, W2, b2):
    src = edge_index[0]
    dst = edge_index[1]
    deg_f = jnp.zeros((N,), jnp.float32).at[dst].add(edge_weight)
    deg_b = jnp.zeros((N,), jnp.float32).at[src].add(edge_weight)
    wf = edge_weight / jnp.where(deg_f == 0, 1.0, deg_f)[dst]
    wb = edge_weight / jnp.where(deg_b == 0, 1.0, deg_b)[src]

    pad = EP - E
    zi = jnp.zeros((pad,), jnp.int32)
    zf = jnp.zeros((pad,), jnp.float32)
    src_p = jnp.concatenate([src, zi])
    dst_p = jnp.concatenate([dst, zi])
    gi = jnp.stack([src_p, dst_p]).reshape(2, NTILES, CHUNKS, CW)
    si = jnp.stack([dst_p, src_p]).reshape(2, NTILES, CHUNKS, CW)
    we = jnp.stack([jnp.concatenate([wf, zf]),
                    jnp.concatenate([wb, zf])]).reshape(2, NTILES, CHUNKS, CW)

    Wx_r, Wh_r = _split_w(Wr)
    Wx_u, Wh_u = _split_w(Wu)
    Wx_c, Wh_c = _split_w(Wc)
    Wx_all = jnp.concatenate([Wx_r, Wx_u, Wx_c], axis=1)
    b_all = jnp.concatenate([br, bu, bc])
    Wh_ru = jnp.concatenate([Wh_r, Wh_u], axis=1)

    xe = _mm(x[0].reshape(T * N, -1), W_enc, b_enc).reshape(T, N, H)

    xcat = _chain_cat(xe, gi, si, we, _chain_sc_T)
    ct = _mm(xcat.reshape(T * N, 160), Wx_all, b_all).reshape(T, N, 96)

    h = jnp.zeros((N, H), jnp.float32)
    for t in range(T):
        hcat = _chain_cat(h[None], gi, si, we, _chain_sc_1)[0]
        g, u = _gates_ru(ct[t, :, :64], hcat, Wh_ru, h)
        gcat = _chain_cat(g[None], gi, si, we, _chain_sc_1)[0]
        h = _gates_c(ct[t, :, 64:], gcat, Wh_c, u, h)

    z = _decoder(h, W1, b1, W2, b2)
    out = z.reshape(1, N, 12, 128).transpose(0, 2, 1, 3)
    return out
